# baseline (speedup 1.0000x reference)
import jax
import jax.numpy as jnp
from jax import lax
from jax.experimental import pallas as pl
from jax.experimental.pallas import tpu as pltpu
from jax.experimental.pallas import tpu_sc as plsc

_N = 10000
_F = 128
_E = 320000
_NC = 2
_NS = 16
_L = 16
_NW = _NC * _NS
_EPW = _E // _NW
_K = 80
_NCH = _EPW // _K
_NP = 10240
_RPT = _NP // _NS


def _set_idx(idxv, base):
    for g in range(_K // _L):
        idxv[pl.ds(g * _L, _L)] = lax.iota(jnp.int32, _L) + (base + g * _L)


def _zero_rows(rows):
    zero16 = jnp.zeros((_L,), jnp.float32)

    def _zb(r, carry):
        for f in range(_F // _L):
            rows[r, pl.ds(f * _L, _L)] = zero16
        return carry

    lax.fori_loop(0, _K, _zb, 0)


def _zero_acc(acc, idxv, rows, s):
    for i in range(_RPT // _K):
        _set_idx(idxv, s * _RPT + i * _K)
        pltpu.sync_copy(rows, acc.at[idxv])


def _acc_to_hbm(acc, idxv, rows, out, c, s):
    for i in range(_RPT // _K):
        r0 = s * _RPT + i * _K
        _set_idx(idxv, r0)
        pltpu.sync_copy(acc.at[idxv], rows)
        pltpu.sync_copy(rows, out.at[c, pl.ds(r0, _K)])


_BLK = 400
_CPB = _BLK // _K
_NBLK = _EPW // _BLK


def _sc_layer_body(h, srcr, dstr, wr, outm,
                   src_all, dst_all, w_all, dstv, idxv, rows, accm, semg,
                   sema):
    c = lax.axis_index("c")
    s = lax.axis_index("s")
    wid = s * _NC + c

    zero16 = jnp.zeros((_L,), jnp.float32)

    def _zb(r, carry):
        for f in range(_F // _L):
            rows[0, r, pl.ds(f * _L, _L)] = zero16
        return carry

    lax.fori_loop(0, _K, _zb, 0)
    for i in range(_RPT // _K):
        _set_idx(idxv, s * _RPT + i * _K)
        pltpu.sync_copy(rows.at[0], accm.at[idxv])
    plsc.subcore_barrier()

    def _gather(g, b):
        return pltpu.async_copy(
            h.at[src_all.at[pl.ds(g * _K, _K)]], rows.at[b], semg.at[b])

    for blk in range(_NBLK):
        eb = wid * _EPW + blk * _BLK
        pltpu.sync_copy(srcr.at[pl.ds(eb, _BLK)], src_all)
        pltpu.sync_copy(dstr.at[pl.ds(eb, _BLK)], dst_all)
        pltpu.sync_copy(wr.at[pl.ds(eb, _BLK)], w_all)
        _gather(0, 0)
        _gather(1, 1)

        def _chunk(i, carry):
            b = lax.rem(i, 3)
            pltpu.make_async_copy(
                h.at[src_all.at[pl.ds(i * _K, _K)]], rows.at[b], semg.at[b]
            ).wait()

            @plsc.parallel_loop(0, _K, unroll=4)
            def _scale(e):
                wb = plsc.load_gather(
                    w_all, [jnp.full((_L,), i * _K + e, jnp.int32)])
                for f in range(_F // _L):
                    rows[b, e, pl.ds(f * _L, _L)] = (
                        rows[b, e, pl.ds(f * _L, _L)] * wb)
            for g2 in range(_K // _L):
                dstv[b, pl.ds(g2 * _L, _L)] = dst_all[
                    pl.ds(i * _K + g2 * _L, _L)]
            pltpu.async_copy(rows.at[b], accm.at[dstv.at[b]], sema.at[b],
                             add=True)

            @pl.when(i + 2 < _CPB)
            def _():
                b2 = lax.rem(i + 2, 3)

                @pl.when(i >= 1)
                def _():
                    pltpu.make_async_copy(
                        rows.at[b2], accm.at[dstv.at[b2]], sema.at[b2]).wait()

                _gather(i + 2, b2)

            return carry

        lax.fori_loop(0, _CPB, _chunk, 0)
        for bb in ((_CPB - 3) % 3, (_CPB - 2) % 3, (_CPB - 1) % 3):
            pltpu.make_async_copy(
                rows.at[bb], accm.at[dstv.at[bb]], sema.at[bb]).wait()

    plsc.subcore_barrier()
    _acc_to_hbm(accm, idxv, rows.at[0], outm, c, s)


def _sc_scal_body(dstr, wr, outs, dst_all, w_all, dstv2, idxv, rowsW, accw,
                  sema):
    c = lax.axis_index("c")
    s = lax.axis_index("s")
    wid = s * _NC + c

    zero16 = jnp.zeros((_L,), jnp.float32)

    def _zb(r, carry):
        for bb in range(2):
            for f in range(_F // _L):
                rowsW[bb, r, pl.ds(f * _L, _L)] = zero16
        return carry

    lax.fori_loop(0, _K, _zb, 0)
    _zero_acc(accw, idxv, rowsW.at[0], s)
    plsc.subcore_barrier()

    ones16 = jnp.ones((_L,), jnp.float32)
    col0 = jnp.zeros((_L,), jnp.int32)
    col1 = jnp.full((_L,), 1, jnp.int32)

    for blk in range(_NBLK):
        eb = wid * _EPW + blk * _BLK
        pltpu.sync_copy(dstr.at[pl.ds(eb, _BLK)], dst_all)
        pltpu.sync_copy(wr.at[pl.ds(eb, _BLK)], w_all)

        def _chunk(i, carry):
            b = lax.rem(i, 2)

            @pl.when(i >= 2)
            def _():
                pltpu.make_async_copy(
                    rowsW.at[b], accw.at[dstv2.at[b]], sema.at[b]).wait()

            for g in range(_K // _L):
                w16 = w_all[pl.ds(i * _K + g * _L, _L)]
                ridx = lax.iota(jnp.int32, _L) + g * _L
                plsc.store_scatter(rowsW.at[b], [ridx, col0], w16)
                plsc.store_scatter(rowsW.at[b], [ridx, col1], ones16)
                dstv2[b, pl.ds(g * _L, _L)] = dst_all[
                    pl.ds(i * _K + g * _L, _L)]
            pltpu.async_copy(rowsW.at[b], accw.at[dstv2.at[b]], sema.at[b],
                             add=True)
            return carry

        lax.fori_loop(0, _CPB, _chunk, 0)
        for bb in range(2):
            pltpu.make_async_copy(
                rowsW.at[bb], accw.at[dstv2.at[bb]], sema.at[bb]).wait()

    plsc.subcore_barrier()
    _acc_to_hbm(accw, idxv, rowsW.at[0], outs, c, s)


def _make_sc_layer():
    mesh = plsc.VectorSubcoreMesh(core_axis_name="c", subcore_axis_name="s",
                                  num_cores=_NC, num_subcores=_NS)
    return pl.kernel(
        _sc_layer_body,
        out_type=jax.ShapeDtypeStruct((_NC, _NP, _F), jnp.float32),
        mesh=mesh,
        scratch_types=[
            pltpu.VMEM((_BLK,), jnp.int32),
            pltpu.VMEM((_BLK,), jnp.int32),
            pltpu.VMEM((_BLK,), jnp.float32),
            pltpu.VMEM((3, _K), jnp.int32),
            pltpu.VMEM((_K,), jnp.int32),
            pltpu.VMEM((3, _K, _F), jnp.float32),
            pltpu.VMEM_SHARED((_NP, _F), jnp.float32),
            pltpu.SemaphoreType.DMA((3,)),
            pltpu.SemaphoreType.DMA((3,)),
        ],
        compiler_params=pltpu.CompilerParams(needs_layout_passes=False),
    )


def _make_sc_scal():
    mesh = plsc.VectorSubcoreMesh(core_axis_name="c", subcore_axis_name="s",
                                  num_cores=_NC, num_subcores=_NS)
    return pl.kernel(
        _sc_scal_body,
        out_type=jax.ShapeDtypeStruct((_NC, _NP, _F), jnp.float32),
        mesh=mesh,
        scratch_types=[
            pltpu.VMEM((_BLK,), jnp.int32),
            pltpu.VMEM((_BLK,), jnp.float32),
            pltpu.VMEM((2, _K), jnp.int32),
            pltpu.VMEM((_K,), jnp.int32),
            pltpu.VMEM((2, _K, _F), jnp.float32),
            pltpu.VMEM_SHARED((_NP, _F), jnp.float32),
            pltpu.SemaphoreType.DMA((2,)),
        ],
        compiler_params=pltpu.CompilerParams(needs_layout_passes=False),
    )


_sc_layer = _make_sc_layer()
_sc_scal = _make_sc_scal()


def _combine(s_ref, h_ref, summ):
    ssc = s_ref[0] + s_ref[1]
    sw = ssc[:, 0:1]
    deg = ssc[:, 1:2]
    denom = (deg + 1.0) * sw
    safe = jnp.where(denom == 0.0, 1.0, denom)
    agg = jnp.where(deg > 0.0, deg * summ / safe, 0.0)
    return agg + h_ref[...] / (deg + 1.0)


def _combine1_body(p_ref, s_ref, h_ref, w_ref, b_ref, o_ref):
    neigh = _combine(s_ref, h_ref, p_ref[0] + p_ref[1])
    o_ref[...] = (jnp.dot(neigh, w_ref[...], preferred_element_type=jnp.float32)
                  + b_ref[...])


def _combine2_body(p_ref, s_ref, h_ref, w_ref, b_ref, wo_ref, bo_ref, o_ref):
    neigh = _combine(s_ref, h_ref, p_ref[0] + p_ref[1])
    h2 = (jnp.dot(neigh, w_ref[...], preferred_element_type=jnp.float32)
          + b_ref[...])
    o_ref[...] = (jnp.dot(h2, wo_ref[...], preferred_element_type=jnp.float32)
                  + bo_ref[...])


_BN = 1000
_NCLS = 64


def _tc_combine1(pm, ps, h, Wt, b):
    return pl.pallas_call(
        _combine1_body,
        grid=(_N // _BN,),
        in_specs=[
            pl.BlockSpec((_NC, _BN, _F), lambda i: (0, i, 0)),
            pl.BlockSpec((_NC, _BN, _F), lambda i: (0, i, 0)),
            pl.BlockSpec((_BN, _F), lambda i: (i, 0)),
            pl.BlockSpec((_F, _F), lambda i: (0, 0)),
            pl.BlockSpec((1, _F), lambda i: (0, 0)),
        ],
        out_specs=pl.BlockSpec((_BN, _F), lambda i: (i, 0)),
        out_shape=jax.ShapeDtypeStruct((_N, _F), jnp.float32),
    )(pm, ps, h, Wt, b)


def _tc_combine2(pm, ps, h, Wt, b, Wot, bo):
    return pl.pallas_call(
        _combine2_body,
        grid=(_N // _BN,),
        in_specs=[
            pl.BlockSpec((_NC, _BN, _F), lambda i: (0, i, 0)),
            pl.BlockSpec((_NC, _BN, _F), lambda i: (0, i, 0)),
            pl.BlockSpec((_BN, _F), lambda i: (i, 0)),
            pl.BlockSpec((_F, _F), lambda i: (0, 0)),
            pl.BlockSpec((1, _F), lambda i: (0, 0)),
            pl.BlockSpec((_F, _NCLS), lambda i: (0, 0)),
            pl.BlockSpec((1, _NCLS), lambda i: (0, 0)),
        ],
        out_specs=pl.BlockSpec((_BN, _NCLS), lambda i: (i, 0)),
        out_shape=jax.ShapeDtypeStruct((_N, _NCLS), jnp.float32),
    )(pm, ps, h, Wt, b, Wot, bo)


def kernel(x, edge_index, edge_weight, W1, b1, W2, b2, Wout, bout):
    src = edge_index[0].astype(jnp.int32)
    dst = edge_index[1].astype(jnp.int32)
    w = edge_weight.astype(jnp.float32)

    ps = _sc_scal(dst, w)
    pm1 = _sc_layer(x, src, dst, w)
    h1 = _tc_combine1(pm1, ps, x, W1.T, b1.reshape(1, _F))
    pm2 = _sc_layer(h1, src, dst, w)
    out = _tc_combine2(pm2, ps, h1, W2.T, b2.reshape(1, _F),
                       Wout.T, bout.reshape(1, _NCLS))
    return out


# speedup vs baseline: 1.2304x; 1.2304x over previous
import jax
import jax.numpy as jnp
from jax import lax
from jax.experimental import pallas as pl
from jax.experimental.pallas import tpu as pltpu
from jax.experimental.pallas import tpu_sc as plsc

_N = 10000
_F = 128
_E = 320000
_NC = 2
_NS = 16
_L = 16
_NW = _NC * _NS
_EPW = _E // _NW
_K = 80
_NCH = _EPW // _K
_NP = 10240
_RPT = _NP // _NS


def _set_idx(idxv, base):
    for g in range(_K // _L):
        idxv[pl.ds(g * _L, _L)] = lax.iota(jnp.int32, _L) + (base + g * _L)


def _zero_rows(rows):
    zero16 = jnp.zeros((_L,), jnp.float32)

    def _zb(r, carry):
        for f in range(_F // _L):
            rows[r, pl.ds(f * _L, _L)] = zero16
        return carry

    lax.fori_loop(0, _K, _zb, 0)


def _zero_acc(acc, idxv, rows, s):
    for i in range(_RPT // _K):
        _set_idx(idxv, s * _RPT + i * _K)
        pltpu.sync_copy(rows, acc.at[idxv])


def _acc_to_hbm(acc, idxv, rows, out, c, s):
    for i in range(_RPT // _K):
        r0 = s * _RPT + i * _K
        _set_idx(idxv, r0)
        pltpu.sync_copy(acc.at[idxv], rows)
        pltpu.sync_copy(rows, out.at[c, pl.ds(r0, _K)])


_BLK = 2000
_CPB = _BLK // _K
_NBLK = _EPW // _BLK


def _sc_layer_body(h, srcr, dstr, wr, outm,
                   src_all, dst_all, w_all, dstv, idxv, rows, accm, semg):
    c = lax.axis_index("c")
    s = lax.axis_index("s")
    wid = s * _NC + c

    zero16 = jnp.zeros((_L,), jnp.float32)

    def _zb(r, carry):
        for f in range(_F // _L):
            rows[0, r, pl.ds(f * _L, _L)] = zero16
        return carry

    lax.fori_loop(0, _K, _zb, 0)
    for i in range(_RPT // _K):
        _set_idx(idxv, s * _RPT + i * _K)
        pltpu.sync_copy(rows.at[0], accm.at[idxv])
    plsc.subcore_barrier()

    def _gather(g, b):
        return pltpu.async_copy(
            h.at[src_all.at[pl.ds(g * _K, _K)]], rows.at[b], semg.at[b])

    for blk in range(_NBLK):
        eb = wid * _EPW + blk * _BLK
        pltpu.sync_copy(srcr.at[pl.ds(eb, _BLK)], src_all)
        pltpu.sync_copy(dstr.at[pl.ds(eb, _BLK)], dst_all)
        pltpu.sync_copy(wr.at[pl.ds(eb, _BLK)], w_all)
        _gather(0, 0)
        _gather(1, 1)

        def _chunk(i, carry):
            b = lax.rem(i, 2)
            pltpu.make_async_copy(
                h.at[src_all.at[pl.ds(i * _K, _K)]], rows.at[b], semg.at[b]
            ).wait()

            @plsc.parallel_loop(0, _K, unroll=4)
            def _scale(e):
                wb = plsc.load_gather(
                    w_all, [jnp.full((_L,), i * _K + e, jnp.int32)])
                for f in range(_F // _L):
                    rows[b, e, pl.ds(f * _L, _L)] = (
                        rows[b, e, pl.ds(f * _L, _L)] * wb)
            for g2 in range(_K // _L):
                dstv[pl.ds(g2 * _L, _L)] = dst_all[pl.ds(i * _K + g2 * _L, _L)]
            pltpu.sync_copy(rows.at[b], accm.at[dstv], add=True)

            @pl.when(i + 2 < _CPB)
            def _():
                _gather(i + 2, b)

            return carry

        lax.fori_loop(0, _CPB, _chunk, 0)

    plsc.subcore_barrier()
    _acc_to_hbm(accm, idxv, rows.at[0], outm, c, s)


def _sc_scal_body(dstr, wr, outs, dst_all, w_all, dstv2, idxv, rowsW, accw,
                  sema):
    c = lax.axis_index("c")
    s = lax.axis_index("s")
    wid = s * _NC + c

    zero16 = jnp.zeros((_L,), jnp.float32)

    def _zb(r, carry):
        for bb in range(2):
            for f in range(_F // _L):
                rowsW[bb, r, pl.ds(f * _L, _L)] = zero16
        return carry

    lax.fori_loop(0, _K, _zb, 0)
    _zero_acc(accw, idxv, rowsW.at[0], s)
    plsc.subcore_barrier()

    ones16 = jnp.ones((_L,), jnp.float32)
    col0 = jnp.zeros((_L,), jnp.int32)
    col1 = jnp.full((_L,), 1, jnp.int32)

    for blk in range(_NBLK):
        eb = wid * _EPW + blk * _BLK
        pltpu.sync_copy(dstr.at[pl.ds(eb, _BLK)], dst_all)
        pltpu.sync_copy(wr.at[pl.ds(eb, _BLK)], w_all)

        def _chunk(i, carry):
            b = lax.rem(i, 2)

            @pl.when(i >= 2)
            def _():
                pltpu.make_async_copy(
                    rowsW.at[b], accw.at[dstv2.at[b]], sema.at[b]).wait()

            for g in range(_K // _L):
                w16 = w_all[pl.ds(i * _K + g * _L, _L)]
                ridx = lax.iota(jnp.int32, _L) + g * _L
                plsc.store_scatter(rowsW.at[b], [ridx, col0], w16)
                plsc.store_scatter(rowsW.at[b], [ridx, col1], ones16)
                dstv2[b, pl.ds(g * _L, _L)] = dst_all[
                    pl.ds(i * _K + g * _L, _L)]
            pltpu.async_copy(rowsW.at[b], accw.at[dstv2.at[b]], sema.at[b],
                             add=True)
            return carry

        lax.fori_loop(0, _CPB, _chunk, 0)
        for bb in range(2):
            pltpu.make_async_copy(
                rowsW.at[bb], accw.at[dstv2.at[bb]], sema.at[bb]).wait()

    plsc.subcore_barrier()
    _acc_to_hbm(accw, idxv, rowsW.at[0], outs, c, s)


def _make_sc_layer():
    mesh = plsc.VectorSubcoreMesh(core_axis_name="c", subcore_axis_name="s",
                                  num_cores=_NC, num_subcores=_NS)
    return pl.kernel(
        _sc_layer_body,
        out_type=jax.ShapeDtypeStruct((_NC, _NP, _F), jnp.float32),
        mesh=mesh,
        scratch_types=[
            pltpu.VMEM((_BLK,), jnp.int32),
            pltpu.VMEM((_BLK,), jnp.int32),
            pltpu.VMEM((_BLK,), jnp.float32),
            pltpu.VMEM((_K,), jnp.int32),
            pltpu.VMEM((_K,), jnp.int32),
            pltpu.VMEM((2, _K, _F), jnp.float32),
            pltpu.VMEM_SHARED((_NP, _F), jnp.float32),
            pltpu.SemaphoreType.DMA((2,)),
        ],
        compiler_params=pltpu.CompilerParams(needs_layout_passes=False),
    )


def _make_sc_scal():
    mesh = plsc.VectorSubcoreMesh(core_axis_name="c", subcore_axis_name="s",
                                  num_cores=_NC, num_subcores=_NS)
    return pl.kernel(
        _sc_scal_body,
        out_type=jax.ShapeDtypeStruct((_NC, _NP, _F), jnp.float32),
        mesh=mesh,
        scratch_types=[
            pltpu.VMEM((_BLK,), jnp.int32),
            pltpu.VMEM((_BLK,), jnp.float32),
            pltpu.VMEM((2, _K), jnp.int32),
            pltpu.VMEM((_K,), jnp.int32),
            pltpu.VMEM((2, _K, _F), jnp.float32),
            pltpu.VMEM_SHARED((_NP, _F), jnp.float32),
            pltpu.SemaphoreType.DMA((2,)),
        ],
        compiler_params=pltpu.CompilerParams(needs_layout_passes=False),
    )


_sc_layer = _make_sc_layer()
_sc_scal = _make_sc_scal()


def _combine(s_ref, h_ref, summ):
    ssc = s_ref[0] + s_ref[1]
    sw = ssc[:, 0:1]
    deg = ssc[:, 1:2]
    denom = (deg + 1.0) * sw
    safe = jnp.where(denom == 0.0, 1.0, denom)
    agg = jnp.where(deg > 0.0, deg * summ / safe, 0.0)
    return agg + h_ref[...] / (deg + 1.0)


def _combine1_body(p_ref, s_ref, h_ref, w_ref, b_ref, o_ref):
    neigh = _combine(s_ref, h_ref, p_ref[0] + p_ref[1])
    o_ref[...] = (jnp.dot(neigh, w_ref[...], preferred_element_type=jnp.float32)
                  + b_ref[...])


def _combine2_body(p_ref, s_ref, h_ref, w_ref, b_ref, wo_ref, bo_ref, o_ref):
    neigh = _combine(s_ref, h_ref, p_ref[0] + p_ref[1])
    h2 = (jnp.dot(neigh, w_ref[...], preferred_element_type=jnp.float32)
          + b_ref[...])
    o_ref[...] = (jnp.dot(h2, wo_ref[...], preferred_element_type=jnp.float32)
                  + bo_ref[...])


_BN = 1000
_NCLS = 64


def _tc_combine1(pm, ps, h, Wt, b):
    return pl.pallas_call(
        _combine1_body,
        grid=(_N // _BN,),
        in_specs=[
            pl.BlockSpec((_NC, _BN, _F), lambda i: (0, i, 0)),
            pl.BlockSpec((_NC, _BN, _F), lambda i: (0, i, 0)),
            pl.BlockSpec((_BN, _F), lambda i: (i, 0)),
            pl.BlockSpec((_F, _F), lambda i: (0, 0)),
            pl.BlockSpec((1, _F), lambda i: (0, 0)),
        ],
        out_specs=pl.BlockSpec((_BN, _F), lambda i: (i, 0)),
        out_shape=jax.ShapeDtypeStruct((_N, _F), jnp.float32),
    )(pm, ps, h, Wt, b)


def _tc_combine2(pm, ps, h, Wt, b, Wot, bo):
    return pl.pallas_call(
        _combine2_body,
        grid=(_N // _BN,),
        in_specs=[
            pl.BlockSpec((_NC, _BN, _F), lambda i: (0, i, 0)),
            pl.BlockSpec((_NC, _BN, _F), lambda i: (0, i, 0)),
            pl.BlockSpec((_BN, _F), lambda i: (i, 0)),
            pl.BlockSpec((_F, _F), lambda i: (0, 0)),
            pl.BlockSpec((1, _F), lambda i: (0, 0)),
            pl.BlockSpec((_F, _NCLS), lambda i: (0, 0)),
            pl.BlockSpec((1, _NCLS), lambda i: (0, 0)),
        ],
        out_specs=pl.BlockSpec((_BN, _NCLS), lambda i: (i, 0)),
        out_shape=jax.ShapeDtypeStruct((_N, _NCLS), jnp.float32),
    )(pm, ps, h, Wt, b, Wot, bo)


def kernel(x, edge_index, edge_weight, W1, b1, W2, b2, Wout, bout):
    src = edge_index[0].astype(jnp.int32)
    dst = edge_index[1].astype(jnp.int32)
    w = edge_weight.astype(jnp.float32)

    ps = _sc_scal(dst, w)
    pm1 = _sc_layer(x, src, dst, w)
    h1 = _tc_combine1(pm1, ps, x, W1.T, b1.reshape(1, _F))
    pm2 = _sc_layer(h1, src, dst, w)
    out = _tc_combine2(pm2, ps, h1, W2.T, b2.reshape(1, _F),
                       Wout.T, bout.reshape(1, _NCLS))
    return out
